# speedup vs baseline: 1.2144x; 1.2144x over previous
import math

import jax
import jax.numpy as jnp
from jax import lax
from jax.experimental import pallas as pl
from jax.experimental.pallas import tpu as pltpu
from jax.experimental.pallas import tpu_sc as plsc

D_MODEL = 768
SCALE = math.sqrt(D_MODEL)
LANES = 16

NUM_CORES = 2
NUM_SUBCORES = 16
NW = NUM_CORES * NUM_SUBCORES

B_TOTAL = 4 * 8192
BPW = B_TOTAL // NW
CHUNK = 64
NCHUNK = BPW // CHUNK


def _body(x_hbm, table_hbm, out_hbm, idx_v, rows_v, sem):
    wid = lax.axis_index("s") * NUM_CORES + lax.axis_index("c")
    base = wid * BPW

    pltpu.sync_copy(x_hbm.at[wid], idx_v)

    @pl.loop(0, NCHUNK)
    def chunk(j):
        pltpu.async_copy(table_hbm.at[idx_v.at[j]], rows_v, sem).wait()

        @pl.loop(0, CHUNK)
        def row(r):
            for d in range(D_MODEL // LANES):
                sl = pl.ds(d * LANES, LANES)
                rows_v[r, sl] = rows_v[r, sl] * SCALE

        pltpu.sync_copy(rows_v, out_hbm.at[pl.ds(base + j * CHUNK, CHUNK)])


def _make_kernel():
    mesh = plsc.VectorSubcoreMesh(
        core_axis_name="c", subcore_axis_name="s",
        num_cores=NUM_CORES, num_subcores=NUM_SUBCORES,
    )
    return pl.kernel(
        _body,
        out_type=jax.ShapeDtypeStruct((B_TOTAL, D_MODEL), jnp.float32),
        mesh=mesh,
        scratch_types=[
            pltpu.VMEM((NCHUNK, CHUNK), jnp.int32),
            pltpu.VMEM((CHUNK, D_MODEL), jnp.float32),
            pltpu.SemaphoreType.DMA,
        ],
    )


_lookup = _make_kernel()


def kernel(x, table):
    b, s = x.shape
    x3 = x.astype(jnp.int32).reshape(NW, NCHUNK, CHUNK)
    out = _lookup(x3, table)
    return out.reshape(b, s, D_MODEL)


# speedup vs baseline: 1.6378x; 1.3486x over previous
import math

import jax
import jax.numpy as jnp
from jax import lax
from jax.experimental import pallas as pl
from jax.experimental.pallas import tpu as pltpu
from jax.experimental.pallas import tpu_sc as plsc

D_MODEL = 768
SCALE = math.sqrt(D_MODEL)
LANES = 16

NUM_CORES = 2
NUM_SUBCORES = 16
NW = NUM_CORES * NUM_SUBCORES

B_TOTAL = 4 * 8192
BPW = B_TOTAL // NW
CHUNK = 64
NCHUNK = BPW // CHUNK


def _body(x_hbm, table_hbm, out_hbm, idx_v, rows0, rows1, sg0, sg1, so0, so1):
    wid = lax.axis_index("s") * NUM_CORES + lax.axis_index("c")
    base = wid * BPW
    bufs = (rows0, rows1)
    gsems = (sg0, sg1)
    osems = (so0, so1)

    pltpu.sync_copy(x_hbm.at[wid], idx_v)

    def gather_cp(j, b):
        return pltpu.make_async_copy(table_hbm.at[idx_v.at[j]], bufs[b], gsems[b])

    def out_cp(j, b):
        dst = out_hbm.at[pl.ds(base + j * CHUNK, CHUNK)]
        return pltpu.make_async_copy(bufs[b], dst, osems[b])

    gather_cp(0, 0).start()

    @pl.loop(0, NCHUNK, step=2)
    def outer(g):
        for b in range(2):
            j = g + b
            other = 1 - b

            @pl.when(j >= 1)
            def _wait_prev_write():
                out_cp(j - 1, other).wait()

            @pl.when(j + 1 < NCHUNK)
            def _prefetch():
                gather_cp(j + 1, other).start()

            gather_cp(j, b).wait()

            @pl.loop(0, CHUNK)
            def row(r):
                for d in range(D_MODEL // LANES):
                    sl = pl.ds(d * LANES, LANES)
                    bufs[b][r, sl] = bufs[b][r, sl] * SCALE

            out_cp(j, b).start()

    out_cp(NCHUNK - 1, 1).wait()


def _make_kernel():
    mesh = plsc.VectorSubcoreMesh(
        core_axis_name="c", subcore_axis_name="s",
        num_cores=NUM_CORES, num_subcores=NUM_SUBCORES,
    )
    return pl.kernel(
        _body,
        out_type=jax.ShapeDtypeStruct((B_TOTAL, D_MODEL), jnp.float32),
        mesh=mesh,
        scratch_types=[
            pltpu.VMEM((NCHUNK, CHUNK), jnp.int32),
            pltpu.VMEM((CHUNK, D_MODEL), jnp.float32),
            pltpu.VMEM((CHUNK, D_MODEL), jnp.float32),
            pltpu.SemaphoreType.DMA,
            pltpu.SemaphoreType.DMA,
            pltpu.SemaphoreType.DMA,
            pltpu.SemaphoreType.DMA,
        ],
    )


_lookup = _make_kernel()


def kernel(x, table):
    b, s = x.shape
    x3 = x.astype(jnp.int32).reshape(NW, NCHUNK, CHUNK)
    out = _lookup(x3, table)
    return out.reshape(b, s, D_MODEL)
